# Initial kernel scaffold; baseline (speedup 1.0000x reference)
import functools

import jax
import jax.numpy as jnp
from jax import lax
from jax.experimental import pallas as pl
from jax.experimental.pallas import tpu as pltpu
from jax.experimental.pallas import tpu_sc as plsc

B = 16384
NNUM = 13
NCAT = 26
VOCAB = 100000
EDIM = 16
L1 = 200
L2 = 50
NCLS = 2

_NC = 2
_NS = 16
_NW = _NC * _NS
_ROWS = B * NCAT
_RPW = _ROWS // _NW
_CHUNK = 3328
_NCHUNK = _RPW // _CHUNK


def _sc_gather(flat_table, gidx):
    mesh = plsc.VectorSubcoreMesh(
        core_axis_name="c", subcore_axis_name="s",
        num_cores=_NC, num_subcores=_NS)

    @functools.partial(
        pl.kernel,
        out_type=jax.ShapeDtypeStruct((_ROWS, EDIM), jnp.float32),
        mesh=mesh,
        scratch_types=[
            pltpu.VMEM((_RPW,), jnp.int32),
            pltpu.VMEM((_CHUNK, EDIM), jnp.float32),
            pltpu.SemaphoreType.DMA,
        ],
    )
    def gather_kernel(flat_hbm, idx_hbm, out_hbm, idx_v, rows_v, sem):
        wid = lax.axis_index("s") * _NC + lax.axis_index("c")
        base = wid * _RPW
        pltpu.sync_copy(idx_hbm.at[pl.ds(base, _RPW)], idx_v)

        def body(i, carry):
            off = i * _CHUNK
            pltpu.async_copy(
                flat_hbm.at[idx_v.at[pl.ds(off, _CHUNK)]], rows_v, sem
            ).wait()
            pltpu.sync_copy(rows_v, out_hbm.at[pl.ds(base + off, _CHUNK)])
            return carry

        lax.fori_loop(0, _NCHUNK, body, 0)

    return gather_kernel(flat_table, gidx)


_BLK = 2048


def _mlp_body(nums_ref, emb_ref, w1n_ref, w1e_ref, b1_ref, w2_ref, b2_ref,
              wp_ref, bp_ref, out_ref):
    x = jnp.dot(nums_ref[...], w1n_ref[...], preferred_element_type=jnp.float32)
    x = x + jnp.dot(emb_ref[...], w1e_ref[...],
                    preferred_element_type=jnp.float32)
    h = jnp.maximum(x + b1_ref[...], 0.0)
    h = jnp.maximum(
        jnp.dot(h, w2_ref[...], preferred_element_type=jnp.float32)
        + b2_ref[...], 0.0)
    logits = (jnp.dot(h, wp_ref[...], preferred_element_type=jnp.float32)
              + bp_ref[...])
    m = jnp.max(logits, axis=-1, keepdims=True)
    e = jnp.exp(logits - m)
    out_ref[...] = e / jnp.sum(e, axis=-1, keepdims=True)


def _tc_mlp(nums, emb, w1n, w1e, b1, w2, b2, wp, bp, interpret=False):
    fixed = lambda i: (0, 0)
    return pl.pallas_call(
        _mlp_body,
        grid=(B // _BLK,),
        in_specs=[
            pl.BlockSpec((_BLK, NNUM), lambda i: (i, 0)),
            pl.BlockSpec((_BLK, NCAT * EDIM), lambda i: (i, 0)),
            pl.BlockSpec((NNUM, L1), fixed),
            pl.BlockSpec((NCAT * EDIM, L1), fixed),
            pl.BlockSpec((1, L1), fixed),
            pl.BlockSpec((L1, L2), fixed),
            pl.BlockSpec((1, L2), fixed),
            pl.BlockSpec((L2, NCLS), fixed),
            pl.BlockSpec((1, NCLS), fixed),
        ],
        out_specs=pl.BlockSpec((_BLK, NCLS), lambda i: (i, 0)),
        out_shape=jax.ShapeDtypeStruct((B, NCLS), jnp.float32),
        interpret=interpret,
    )(nums, emb, w1n, w1e, b1, w2, b2, wp, bp)


def kernel(nums, cats, tables, W1, b1, W2, b2, Wp, bp):
    flat_table = tables.reshape(NCAT * VOCAB, EDIM)
    offsets = (jnp.arange(NCAT, dtype=jnp.int32) * VOCAB)[None, :]
    gidx = (cats + offsets).reshape(-1)
    emb = _sc_gather(flat_table, gidx).reshape(B, NCAT * EDIM)
    return _tc_mlp(nums, emb, W1[:NNUM], W1[NNUM:], b1.reshape(1, L1),
                   W2, b2.reshape(1, L2), Wp, bp.reshape(1, NCLS))


# speedup vs baseline: 7.7999x; 7.7999x over previous
import functools

import jax
import jax.numpy as jnp
from jax import lax
from jax.experimental import pallas as pl
from jax.experimental.pallas import tpu as pltpu
from jax.experimental.pallas import tpu_sc as plsc

B = 16384
NNUM = 13
NCAT = 26
VOCAB = 100000
EDIM = 16
L1 = 200
L2 = 50
NCLS = 2

_NC = 2
_NS = 16
_NW = _NC * _NS
_ROWS = B * NCAT
_RPW = _ROWS // _NW
_CHUNK = 3328
_NCHUNK = _RPW // _CHUNK


def _sc_gather(flat_table, gidx):
    mesh = plsc.VectorSubcoreMesh(
        core_axis_name="c", subcore_axis_name="s",
        num_cores=_NC, num_subcores=_NS)

    @functools.partial(
        pl.kernel,
        out_type=jax.ShapeDtypeStruct((_ROWS, EDIM), jnp.float32),
        mesh=mesh,
        scratch_types=[
            pltpu.VMEM((_RPW,), jnp.int32),
            pltpu.VMEM((_CHUNK, EDIM), jnp.float32),
            pltpu.SemaphoreType.DMA,
        ],
        compiler_params=pltpu.CompilerParams(use_tc_tiling_on_sc=False),
    )
    def gather_kernel(flat_hbm, idx_hbm, out_hbm, idx_v, rows_v, sem):
        wid = lax.axis_index("s") * _NC + lax.axis_index("c")
        base = wid * _RPW
        pltpu.sync_copy(idx_hbm.at[pl.ds(base, _RPW)], idx_v)

        def body(i, carry):
            off = i * _CHUNK
            pltpu.async_copy(
                flat_hbm.at[idx_v.at[pl.ds(off, _CHUNK)]], rows_v, sem
            ).wait()
            pltpu.sync_copy(rows_v, out_hbm.at[pl.ds(base + off, _CHUNK)])
            return carry

        lax.fori_loop(0, _NCHUNK, body, 0)

    return gather_kernel(flat_table, gidx)


_BLK = 2048


def _mlp_body(nums_ref, emb_ref, w1n_ref, w1e_ref, b1_ref, w2_ref, b2_ref,
              wp_ref, bp_ref, out_ref):
    x = jnp.dot(nums_ref[...], w1n_ref[...], preferred_element_type=jnp.float32)
    x = x + jnp.dot(emb_ref[...], w1e_ref[...],
                    preferred_element_type=jnp.float32)
    h = jnp.maximum(x + b1_ref[...], 0.0)
    h = jnp.maximum(
        jnp.dot(h, w2_ref[...], preferred_element_type=jnp.float32)
        + b2_ref[...], 0.0)
    logits = (jnp.dot(h, wp_ref[...], preferred_element_type=jnp.float32)
              + bp_ref[...])
    m = jnp.max(logits, axis=-1, keepdims=True)
    e = jnp.exp(logits - m)
    out_ref[...] = e / jnp.sum(e, axis=-1, keepdims=True)


def _tc_mlp(nums, emb, w1n, w1e, b1, w2, b2, wp, bp, interpret=False):
    fixed = lambda i: (0, 0)
    return pl.pallas_call(
        _mlp_body,
        grid=(B // _BLK,),
        in_specs=[
            pl.BlockSpec((_BLK, NNUM), lambda i: (i, 0)),
            pl.BlockSpec((_BLK, NCAT * EDIM), lambda i: (i, 0)),
            pl.BlockSpec((NNUM, L1), fixed),
            pl.BlockSpec((NCAT * EDIM, L1), fixed),
            pl.BlockSpec((1, L1), fixed),
            pl.BlockSpec((L1, L2), fixed),
            pl.BlockSpec((1, L2), fixed),
            pl.BlockSpec((L2, NCLS), fixed),
            pl.BlockSpec((1, NCLS), fixed),
        ],
        out_specs=pl.BlockSpec((_BLK, NCLS), lambda i: (i, 0)),
        out_shape=jax.ShapeDtypeStruct((B, NCLS), jnp.float32),
        interpret=interpret,
    )(nums, emb, w1n, w1e, b1, w2, b2, wp, bp)


def kernel(nums, cats, tables, W1, b1, W2, b2, Wp, bp):
    flat_table = tables.reshape(NCAT * VOCAB, EDIM)
    offsets = (jnp.arange(NCAT, dtype=jnp.int32) * VOCAB)[None, :]
    gidx = (cats + offsets).reshape(-1)
    emb = _sc_gather(flat_table, gidx).reshape(B, NCAT * EDIM)
    return _tc_mlp(nums, emb, W1[:NNUM], W1[NNUM:], b1.reshape(1, L1),
                   W2, b2.reshape(1, L2), Wp, bp.reshape(1, NCLS))
